# Initial kernel scaffold; baseline (speedup 1.0000x reference)
import jax
import jax.numpy as jnp
from jax.experimental import pallas as pl


def kernel(feature_ids, feature_ratings, segment_ids, item_ids, user_table, rating_table, item_table, bias):
    raise NotImplementedError("write your pallas kernel here")


# speedup vs baseline: 1.1778x; 1.1778x over previous
import jax
import jax.numpy as jnp
from jax import lax
from jax.experimental import pallas as pl
from jax.experimental.pallas import tpu as pltpu
from jax.experimental.pallas import tpu_sc as plsc

NUM_ITEMS = 1000000
D = 16
TOTAL_TOK = 409600
B = 16384

NC = 2
NS = 16
NW = NC * NS
TOK_PER_TILE = TOTAL_TOK // NW
CHUNK = 1280
N_CHUNKS = TOK_PER_TILE // CHUNK
G = CHUNK // 128
ROWS_PER_TILE = B // NS
RG = ROWS_PER_TILE // 128


def _sc_body(fid, rat, seg, iid, utab, rtab, itab, bias,
             p0, p1,
             fid2d, seg2d, ridx2d, rat_v, urows, rrows, irows, accrows,
             iid2d, bias_v, acc, sem_in, sem_g):
    c = lax.axis_index("c")
    s = lax.axis_index("s")
    w = c * NS + s
    tile_base = w * TOK_PER_TILE

    @pl.loop(0, ROWS_PER_TILE)
    def _zero(i):
        accrows[i] = jnp.zeros((D,), jnp.float32)

    row0 = s * ROWS_PER_TILE
    pltpu.sync_copy(accrows, acc.at[pl.ds(row0, ROWS_PER_TILE)])
    plsc.subcore_barrier()

    @pl.loop(0, N_CHUNKS)
    def _chunk(ch):
        base = tile_base + ch * CHUNK
        descs = []
        for j in range(G):
            descs.append(pltpu.async_copy(
                fid.at[pl.ds(base + j * 128, 128)], fid2d.at[j], sem_in))
            descs.append(pltpu.async_copy(
                seg.at[pl.ds(base + j * 128, 128)], seg2d.at[j], sem_in))
        descs.append(pltpu.async_copy(rat.at[pl.ds(base, CHUNK)], rat_v, sem_in))
        for d in descs:
            d.wait()

        for r in range(G):
            @pl.loop(0, 8)
            def _q(k, r=r):
                v = rat_v[pl.ds(r * 128 + k * 16, 16)]
                ridx2d[r, pl.ds(k * 16, 16)] = ((v - 0.5) * 2.0).astype(jnp.int32)

        gds = []
        for j in range(G):
            gds.append(pltpu.async_copy(
                utab.at[fid2d.at[j]], urows.at[pl.ds(j * 128, 128)], sem_g))
            gds.append(pltpu.async_copy(
                rtab.at[ridx2d.at[j]], rrows.at[pl.ds(j * 128, 128)], sem_g))
        for d in gds:
            d.wait()

        @pl.loop(0, CHUNK)
        def _mul(i):
            urows[i] = urows[i] * rrows[i]

        for j in range(G):
            pltpu.sync_copy(urows.at[pl.ds(j * 128, 128)],
                            acc.at[seg2d.at[j]], add=True)

    plsc.subcore_barrier()

    pltpu.sync_copy(acc.at[pl.ds(row0, ROWS_PER_TILE)], accrows)

    @pl.when(c == 0)
    def _core0():
        descs = [pltpu.async_copy(
            iid.at[pl.ds(row0 + t * 128, 128)], iid2d.at[t], sem_in)
            for t in range(RG)]
        pltpu.sync_copy(bias, bias_v)
        for d in descs:
            d.wait()
        gds = [pltpu.async_copy(
            itab.at[iid2d.at[t]], irows.at[pl.ds(t * 128, 128)], sem_g)
            for t in range(RG)]
        for d in gds:
            d.wait()
        bvec = bias_v[...]

        @pl.loop(0, ROWS_PER_TILE)
        def _add(i):
            accrows[i] = accrows[i] + irows[i] + bvec

        pltpu.sync_copy(accrows, p0.at[pl.ds(row0, ROWS_PER_TILE)])

    @pl.when(c != 0)
    def _core1():
        pltpu.sync_copy(accrows, p1.at[pl.ds(row0, ROWS_PER_TILE)])


_sc_forward = pl.kernel(
    _sc_body,
    out_type=(jax.ShapeDtypeStruct((B, D), jnp.float32),
              jax.ShapeDtypeStruct((B, D), jnp.float32)),
    mesh=plsc.VectorSubcoreMesh(core_axis_name="c", subcore_axis_name="s"),
    scratch_types=[
        pltpu.VMEM((G, 128), jnp.int32),
        pltpu.VMEM((G, 128), jnp.int32),
        pltpu.VMEM((G, 128), jnp.int32),
        pltpu.VMEM((CHUNK,), jnp.float32),
        pltpu.VMEM((CHUNK, D), jnp.float32),
        pltpu.VMEM((CHUNK, D), jnp.float32),
        pltpu.VMEM((ROWS_PER_TILE, D), jnp.float32),
        pltpu.VMEM((ROWS_PER_TILE, D), jnp.float32),
        pltpu.VMEM((RG, 128), jnp.int32),
        pltpu.VMEM((D,), jnp.float32),
        pltpu.VMEM_SHARED((B, D), jnp.float32),
        pltpu.SemaphoreType.DMA,
        pltpu.SemaphoreType.DMA,
    ],
    compiler_params=pltpu.CompilerParams(use_tc_tiling_on_sc=False),
)


def _combine_body(p0_ref, p1_ref, o_ref):
    o_ref[...] = p0_ref[...] + p1_ref[...]


def kernel(feature_ids, feature_ratings, segment_ids, item_ids,
           user_table, rating_table, item_table, bias):
    fid = feature_ids.astype(jnp.int32)
    seg = segment_ids.astype(jnp.int32)
    iid = item_ids.astype(jnp.int32)
    p0, p1 = _sc_forward(fid, feature_ratings, seg, iid,
                         user_table, rating_table, item_table, bias)
    out = pl.pallas_call(
        _combine_body,
        out_shape=jax.ShapeDtypeStruct((B * D // 128, 128), jnp.float32),
    )(p0.reshape(B * D // 128, 128), p1.reshape(B * D // 128, 128))
    return out.reshape(B, D)


# speedup vs baseline: 3.1276x; 2.6554x over previous
import jax
import jax.numpy as jnp
from jax import lax
from jax.experimental import pallas as pl
from jax.experimental.pallas import tpu as pltpu
from jax.experimental.pallas import tpu_sc as plsc

NUM_ITEMS = 1000000
D = 16
TOTAL_TOK = 409600
B = 16384

NC = 2
NS = 16
SEG_HALF = B // NC
NQ = 10
ACC_SEG = SEG_HALF * NQ
TRASH = ACC_SEG
ACC_ROWS = ACC_SEG + 128
ZROWS = ACC_ROWS // NS
CHUNK = 1280
G = CHUNK // 128
SEG_PER_TILE = SEG_HALF // NS
SUB = SEG_PER_TILE // 128


def _sc_body(fid, rat, seg, iid, splits, utab, rtab, itab, bias, out,
             fid2d, seg2d, cidx2d, rat_v, urows, irows, iid2d, rtab_v, bias_v,
             split_v, split_sm, acc, sem_in, sem_g):
    c = lax.axis_index("c")
    s = lax.axis_index("s")

    pltpu.sync_copy(splits, split_v)
    split = jnp.max(split_v[...], axis=0)

    @pl.loop(0, CHUNK, unroll=4)
    def _z(i):
        urows[i] = jnp.zeros((D,), jnp.float32)

    zbase = s * ZROWS
    for t in range(4):
        pltpu.sync_copy(urows, acc.at[pl.ds(zbase + t * CHUNK, CHUNK)])
    pltpu.sync_copy(urows.at[pl.ds(0, ZROWS - 4 * CHUNK)],
                    acc.at[pl.ds(zbase + 4 * CHUNK, ZROWS - 4 * CHUNK)])
    plsc.subcore_barrier()

    start_c = jnp.where(c == 0, 0, split)
    end_c = jnp.where(c == 0, split, TOTAL_TOK)
    n = end_c - start_c
    m = (n + NS - 1) // NS
    w_start = start_c + s * m
    w_end = jnp.minimum(w_start + m, end_c)
    seg_base = c * SEG_HALF

    first_g = w_start // CHUNK
    last_g1 = jnp.where(w_end > w_start,
                        (w_end - 1) // CHUNK + 1,
                        first_g)

    @pl.loop(first_g, last_g1)
    def _chunk(g):
        base = g * CHUNK
        descs = []
        for j in range(G):
            descs.append(pltpu.async_copy(
                fid.at[pl.ds(base + j * 128, 128)], fid2d.at[j], sem_in))
            descs.append(pltpu.async_copy(
                seg.at[pl.ds(base + j * 128, 128)], seg2d.at[j], sem_in))
        descs.append(pltpu.async_copy(rat.at[pl.ds(base, CHUNK)], rat_v, sem_in))
        for d in descs:
            d.wait()

        lane = lax.iota(jnp.int32, 16)
        for r in range(G):
            @pl.loop(0, 8, unroll=4)
            def _q(k, r=r):
                off = r * 128 + k * 16
                pos = base + off + lane
                sv = seg2d[r, pl.ds(k * 16, 16)]
                rv = rat_v[pl.ds(off, 16)]
                q = ((rv - 0.5) * 2.0).astype(jnp.int32)
                cidx = (sv - seg_base) * NQ + q
                valid = (pos >= w_start) & (pos < w_end)
                cidx2d[r, pl.ds(k * 16, 16)] = jnp.where(valid, cidx, TRASH)

        gds = [pltpu.async_copy(
            utab.at[fid2d.at[j]], urows.at[pl.ds(j * 128, 128)], sem_g)
            for j in range(G)]
        for d in gds:
            d.wait()
        for j in range(G):
            pltpu.sync_copy(urows.at[pl.ds(j * 128, 128)],
                            acc.at[cidx2d.at[j]], add=True)

    plsc.subcore_barrier()

    pltpu.sync_copy(rtab, rtab_v)
    pltpu.sync_copy(bias, bias_v)
    rtv = [rtab_v[q] for q in range(NQ)]
    bvec = bias_v[...]

    for sb in range(SUB):
        srow0 = s * SEG_PER_TILE + sb * 128
        pltpu.sync_copy(acc.at[pl.ds(srow0 * NQ, CHUNK)], urows)
        pltpu.async_copy(iid.at[pl.ds(seg_base + srow0, 128)],
                         iid2d.at[0], sem_in).wait()
        pltpu.async_copy(itab.at[iid2d.at[0]], irows, sem_g).wait()

        @pl.loop(0, 128, unroll=2)
        def _comb(i):
            abase = i * NQ
            v = irows[i] + bvec
            for q in range(NQ):
                v = v + urows[abase + q] * rtv[q]
            irows[i] = v

        pltpu.sync_copy(irows, out.at[pl.ds(seg_base + srow0, 128)])


_sc_forward = pl.kernel(
    _sc_body,
    out_type=jax.ShapeDtypeStruct((B, D), jnp.float32),
    mesh=plsc.VectorSubcoreMesh(core_axis_name="c", subcore_axis_name="s"),
    scratch_types=[
        pltpu.VMEM((G, 128), jnp.int32),
        pltpu.VMEM((G, 128), jnp.int32),
        pltpu.VMEM((G, 128), jnp.int32),
        pltpu.VMEM((CHUNK,), jnp.float32),
        pltpu.VMEM((CHUNK, D), jnp.float32),
        pltpu.VMEM((128, D), jnp.float32),
        pltpu.VMEM((1, 128), jnp.int32),
        pltpu.VMEM((NQ, D), jnp.float32),
        pltpu.VMEM((D,), jnp.float32),
        pltpu.VMEM((16,), jnp.int32),
        pltpu.SMEM((16,), jnp.int32),
        pltpu.VMEM_SHARED((ACC_ROWS, D), jnp.float32),
        pltpu.SemaphoreType.DMA,
        pltpu.SemaphoreType.DMA,
    ],
    compiler_params=pltpu.CompilerParams(use_tc_tiling_on_sc=False,
                                         needs_layout_passes=False),
)


def kernel(feature_ids, feature_ratings, segment_ids, item_ids,
           user_table, rating_table, item_table, bias):
    fid = feature_ids.astype(jnp.int32)
    seg = segment_ids.astype(jnp.int32)
    iid = item_ids.astype(jnp.int32)
    split = jnp.searchsorted(seg, jnp.int32(SEG_HALF)).astype(jnp.int32)
    splits = jnp.full((16,), split, dtype=jnp.int32)
    return _sc_forward(fid, feature_ratings, seg, iid, splits,
                       user_table, rating_table, item_table, bias)


# speedup vs baseline: 4.9615x; 1.5864x over previous
import jax
import jax.numpy as jnp
from jax import lax
from jax.experimental import pallas as pl
from jax.experimental.pallas import tpu as pltpu
from jax.experimental.pallas import tpu_sc as plsc

NUM_ITEMS = 1000000
D = 16
TOTAL_TOK = 409600
B = 16384

NC = 2
NS = 16
SEG_HALF = B // NC
NQ = 10
ACC_SEG = SEG_HALF * NQ
TRASH = ACC_SEG
ACC_ROWS = ACC_SEG + 128
ZROWS = ACC_ROWS // NS
CHUNK = 1280
G = CHUNK // 128
SEG_PER_TILE = SEG_HALF // NS
SUB = SEG_PER_TILE // 128


def _sc_body(fid, rat, seg, splits, utab, rtab, out,
             fid2d, seg2d, cidx2d, rat_v, urows, irows, rtab_v,
             split_v, split_sm, acc, sem_in, sem_g):
    c = lax.axis_index("c")
    s = lax.axis_index("s")

    pltpu.sync_copy(splits, split_v)
    split = jnp.max(split_v[...], axis=0)

    @pl.loop(0, CHUNK, unroll=4)
    def _z(i):
        urows[i] = jnp.zeros((D,), jnp.float32)

    zbase = s * ZROWS
    for t in range(4):
        pltpu.sync_copy(urows, acc.at[pl.ds(zbase + t * CHUNK, CHUNK)])
    pltpu.sync_copy(urows.at[pl.ds(0, ZROWS - 4 * CHUNK)],
                    acc.at[pl.ds(zbase + 4 * CHUNK, ZROWS - 4 * CHUNK)])
    plsc.subcore_barrier()

    start_c = jnp.where(c == 0, 0, split)
    end_c = jnp.where(c == 0, split, TOTAL_TOK)
    n = end_c - start_c
    m = (n + NS - 1) // NS
    w_start = start_c + s * m
    w_end = jnp.minimum(w_start + m, end_c)
    seg_base = c * SEG_HALF

    first_g = w_start // CHUNK
    last_g1 = jnp.where(w_end > w_start,
                        (w_end - 1) // CHUNK + 1,
                        first_g)

    @pl.loop(first_g, last_g1)
    def _chunk(g):
        base = g * CHUNK
        descs = []
        for j in range(G):
            descs.append(pltpu.async_copy(
                fid.at[pl.ds(base + j * 128, 128)], fid2d.at[j], sem_in))
            descs.append(pltpu.async_copy(
                seg.at[pl.ds(base + j * 128, 128)], seg2d.at[j], sem_in))
        descs.append(pltpu.async_copy(rat.at[pl.ds(base, CHUNK)], rat_v, sem_in))
        for d in descs:
            d.wait()

        lane = lax.iota(jnp.int32, 16)
        for r in range(G):
            @pl.loop(0, 8, unroll=4)
            def _q(k, r=r):
                off = r * 128 + k * 16
                pos = base + off + lane
                sv = seg2d[r, pl.ds(k * 16, 16)]
                rv = rat_v[pl.ds(off, 16)]
                q = ((rv - 0.5) * 2.0).astype(jnp.int32)
                cidx = (sv - seg_base) * NQ + q
                valid = (pos >= w_start) & (pos < w_end)
                cidx2d[r, pl.ds(k * 16, 16)] = jnp.where(valid, cidx, TRASH)

        gds = [pltpu.async_copy(
            utab.at[fid2d.at[j]], urows.at[pl.ds(j * 128, 128)], sem_g)
            for j in range(G)]
        for d in gds:
            d.wait()
        for j in range(G):
            pltpu.sync_copy(urows.at[pl.ds(j * 128, 128)],
                            acc.at[cidx2d.at[j]], add=True)

    plsc.subcore_barrier()

    pltpu.sync_copy(rtab, rtab_v)
    rtv = [rtab_v[q] for q in range(NQ)]

    for sb in range(SUB):
        srow0 = s * SEG_PER_TILE + sb * 128
        pltpu.sync_copy(acc.at[pl.ds(srow0 * NQ, CHUNK)], urows)

        @pl.loop(0, 128, unroll=2)
        def _comb(i):
            abase = i * NQ
            v = urows[abase] * rtv[0]
            for q in range(1, NQ):
                v = v + urows[abase + q] * rtv[q]
            irows[i] = v

        pltpu.sync_copy(irows, out.at[pl.ds(seg_base + srow0, 128)])


_sc_forward = pl.kernel(
    _sc_body,
    out_type=jax.ShapeDtypeStruct((B, D), jnp.float32),
    mesh=plsc.VectorSubcoreMesh(core_axis_name="c", subcore_axis_name="s"),
    scratch_types=[
        pltpu.VMEM((G, 128), jnp.int32),
        pltpu.VMEM((G, 128), jnp.int32),
        pltpu.VMEM((G, 128), jnp.int32),
        pltpu.VMEM((CHUNK,), jnp.float32),
        pltpu.VMEM((CHUNK, D), jnp.float32),
        pltpu.VMEM((128, D), jnp.float32),
        pltpu.VMEM((NQ, D), jnp.float32),
        pltpu.VMEM((16,), jnp.int32),
        pltpu.SMEM((16,), jnp.int32),
        pltpu.VMEM_SHARED((ACC_ROWS, D), jnp.float32),
        pltpu.SemaphoreType.DMA,
        pltpu.SemaphoreType.DMA,
    ],
    compiler_params=pltpu.CompilerParams(use_tc_tiling_on_sc=False,
                                         needs_layout_passes=False),
)


def kernel(feature_ids, feature_ratings, segment_ids, item_ids,
           user_table, rating_table, item_table, bias):
    fid = feature_ids.astype(jnp.int32)
    seg = segment_ids.astype(jnp.int32)
    iid = item_ids.astype(jnp.int32)
    split = jnp.sum((seg < SEG_HALF).astype(jnp.int32)).astype(jnp.int32)
    splits = jnp.full((16,), split, dtype=jnp.int32)
    user_sum = _sc_forward(fid, feature_ratings, seg, splits,
                           user_table, rating_table)
    return user_sum + jnp.take(item_table, iid, axis=0) + bias
